# baseline (speedup 1.0000x reference)
import functools

import jax
import jax.numpy as jnp
from jax import lax
from jax.experimental import pallas as pl
from jax.experimental.pallas import tpu as pltpu
from jax.experimental.pallas import tpu_sc as plsc

N = 10000
E = 320000
H = 128
G = 64
CAT = 6

NC = 2
NS = 16
NW = NC * NS
NP = N + 8
EDGES_PER_TILE = E // NW
CHUNK = 64
PADE = 240
EPT_P = EDGES_PER_TILE + PADE
NCHUNK = EPT_P // CHUNK
SUPER = 10
NSUPER = NCHUNK // SUPER
ROWS_MAIN = 624
ROWS_TAIL_OFF = NS * ROWS_MAIN
ROWS_TAIL = NP - ROWS_TAIL_OFF
EXP_TAIL = N - ROWS_TAIL_OFF


def _pre_body(x_ref, w_ref, as_ref, ad_ref, h_ref, a_ref, b_ref, m_ref, exs_ref):
    h = jnp.dot(x_ref[...], w_ref[...], preferred_element_type=jnp.float32)
    h_ref[...] = h
    a = jnp.dot(h, as_ref[...].reshape(H, 1), preferred_element_type=jnp.float32)
    b = jnp.dot(h, ad_ref[...].reshape(H, 1), preferred_element_type=jnp.float32)
    m = jnp.max(a)
    m_ref[...] = jnp.full((1, 1), 0.0, jnp.float32) + m
    cb = m + b
    c = jnp.where(cb > 0, cb, 0.2 * cb)
    z = a + b
    zl = jnp.where(z > 0, z, 0.2 * z)
    a_ref[...] = a
    b_ref[...] = b
    exs_ref[...] = jnp.exp(zl - c)


def _pre_layer(x, W, att_src, att_dst):
    return pl.pallas_call(
        _pre_body,
        out_shape=[
            jax.ShapeDtypeStruct((N, H), jnp.float32),
            jax.ShapeDtypeStruct((N, 1), jnp.float32),
            jax.ShapeDtypeStruct((N, 1), jnp.float32),
            jax.ShapeDtypeStruct((1, 1), jnp.float32),
            jax.ShapeDtypeStruct((N, 1), jnp.float32),
        ],
    )(x, W, att_src, att_dst)


def _post_compute(acc0_ref, acc1_ref, dent_ref, h_ref, exs_ref, bias_ref,
                  gamma_ref, beta_ref):
    exs = exs_ref[...]
    num = acc0_ref[...] + acc1_ref[...] + exs * h_ref[...]
    den = jnp.sum(dent_ref[...], axis=1, keepdims=True) + exs + 1e-16
    y = num / den + bias_ref[...]
    mu = jnp.mean(y, axis=1, keepdims=True)
    yc = y - mu
    var = jnp.mean(yc * yc, axis=1, keepdims=True)
    y = yc * lax.rsqrt(var + 1e-5) * gamma_ref[...] + beta_ref[...]
    return jnp.maximum(y, 0.0)


def _post_pre_body(acc0_ref, acc1_ref, dent_ref, h_ref, exs_ref, bias_ref,
                   gamma_ref, beta_ref, w_ref, as_ref, ad_ref,
                   h1_ref, a_ref, b_ref, m_ref, exs1_ref):
    h1 = _post_compute(acc0_ref, acc1_ref, dent_ref, h_ref, exs_ref,
                       bias_ref, gamma_ref, beta_ref)
    hW = jnp.dot(h1, w_ref[...], preferred_element_type=jnp.float32)
    h1_ref[...] = hW
    a = jnp.dot(hW, as_ref[...].reshape(H, 1),
                preferred_element_type=jnp.float32)
    b = jnp.dot(hW, ad_ref[...].reshape(H, 1),
                preferred_element_type=jnp.float32)
    m = jnp.max(a)
    m_ref[...] = jnp.full((1, 1), 0.0, jnp.float32) + m
    cb = m + b
    c = jnp.where(cb > 0, cb, 0.2 * cb)
    z = a + b
    zl = jnp.where(z > 0, z, 0.2 * z)
    a_ref[...] = a
    b_ref[...] = b
    exs1_ref[...] = jnp.exp(zl - c)


def _post_pre_layer(acc0, acc1, den_t, h, exs, bias, gamma, beta,
                    W, att_src, att_dst):
    return pl.pallas_call(
        _post_pre_body,
        out_shape=[
            jax.ShapeDtypeStruct((N, H), jnp.float32),
            jax.ShapeDtypeStruct((N, 1), jnp.float32),
            jax.ShapeDtypeStruct((N, 1), jnp.float32),
            jax.ShapeDtypeStruct((1, 1), jnp.float32),
            jax.ShapeDtypeStruct((N, 1), jnp.float32),
        ],
    )(acc0, acc1, den_t, h, exs, bias.reshape(1, H), gamma.reshape(1, H),
      beta.reshape(1, H), W, att_src, att_dst)


def _post_final_body(acc0_ref, acc1_ref, dent_ref, h_ref, exs_ref, bias_ref,
                     gamma_ref, beta_ref, batch_ref, cat_ref, wcat_ref,
                     bcat_ref, wlin_ref, blin_ref, out_ref):
    h2 = _post_compute(acc0_ref, acc1_ref, dent_ref, h_ref, exs_ref,
                       bias_ref, gamma_ref, beta_ref)
    bi = batch_ref[...]
    oh = (bi == lax.broadcasted_iota(jnp.int32, (N, G), 1)).astype(jnp.float32)
    s = lax.dot_general(oh, h2, (((0,), (0,)), ((), ())),
                        preferred_element_type=jnp.float32)
    ones = jnp.ones((N, 1), dtype=jnp.float32)
    cnt = lax.dot_general(oh, ones, (((0,), (0,)), ((), ())),
                          preferred_element_type=jnp.float32)
    pooled = s / jnp.maximum(cnt, 1.0)
    cat_emb = jnp.dot(cat_ref[...], wcat_ref[...],
                      preferred_element_type=jnp.float32) + bcat_ref[...]
    cat_emb = jnp.maximum(cat_emb, 0.0)
    z = pooled + cat_emb
    out_ref[...] = jnp.dot(z, wlin_ref[...],
                           preferred_element_type=jnp.float32) + blin_ref[...]


def _post_final(acc0, acc1, den_t, h, exs, bias, gamma, beta, batch,
                cat_features, W_cat, b_cat, W_lin, b_lin):
    return pl.pallas_call(
        _post_final_body,
        out_shape=jax.ShapeDtypeStruct((G, H), jnp.float32),
    )(acc0, acc1, den_t, h, exs, bias.reshape(1, H), gamma.reshape(1, H),
      beta.reshape(1, H), batch.reshape(N, 1), cat_features.reshape(G, CAT),
      W_cat, b_cat.reshape(1, H), W_lin, b_lin.reshape(1, H))


def _sc_body(h_hbm, src_hbm, dst_hbm, a_hbm, b_hbm, m_hbm, zeros_hbm,
             acc_out, den_out,
             a_v, b_v, m_v, den_v, srcs_v, dsts_v, ssrc_v, sdst_v, w_v,
             rowsp_v, rows_v, acc_sh, gath_sem):
    cid = lax.axis_index("c")
    sid = lax.axis_index("s")
    wid = cid * NS + sid

    pltpu.sync_copy(a_hbm, a_v)
    pltpu.sync_copy(b_hbm, b_v)
    pltpu.sync_copy(m_hbm, m_v)

    def _zero_body(i, carry):
        off = pl.multiple_of(i * 16, 16)
        den_v[pl.ds(off, 16)] = jnp.zeros((16,), jnp.float32)
        return carry
    lax.fori_loop(0, NP // 16, _zero_body, 0)
    pltpu.sync_copy(zeros_hbm.at[pl.ds(sid * ROWS_MAIN, ROWS_MAIN)],
                    acc_sh.at[pl.ds(sid * ROWS_MAIN, ROWS_MAIN)])

    @pl.when(sid == 0)
    def _zero_tail():
        pltpu.sync_copy(zeros_hbm.at[pl.ds(ROWS_TAIL_OFF, ROWS_TAIL)],
                        acc_sh.at[pl.ds(ROWS_TAIL_OFF, ROWS_TAIL)])

    plsc.subcore_barrier()

    base = wid * EPT_P
    m16 = m_v[...]
    mask_hi = jnp.full((16,), -65536, jnp.int32)

    def _super_body(i, carry):
        off = pl.multiple_of(base + i * SUPER * CHUNK, 8)
        pltpu.sync_copy(src_hbm.at[pl.ds(off, SUPER * CHUNK)], srcs_v)
        pltpu.sync_copy(dst_hbm.at[pl.ds(off, SUPER * CHUNK)], dsts_v)

        def _chunk_body(ci, carry2):
            for t in range(CHUNK // 16):
                so = pl.multiple_of(ci * CHUNK + t * 16, 16)
                ssrc_v[pl.ds(t * 16, 16)] = srcs_v[pl.ds(so, 16)]
            cp = pltpu.async_copy(h_hbm.at[ssrc_v], rowsp_v, gath_sem)

            for t in range(CHUNK // 16):
                so = pl.multiple_of(ci * CHUNK + t * 16, 16)
                sv = ssrc_v[pl.ds(t * 16, 16)]
                dv = dsts_v[pl.ds(so, 16)]
                ag = plsc.load_gather(a_v, [sv])
                bg = plsc.load_gather(b_v, [dv])
                cb = m16 + bg
                cg = jnp.where(cb > 0, cb, 0.2 * cb)
                z = ag + bg
                e = jnp.where(z > 0, z, 0.2 * z)
                ex = jnp.exp(e - cg)
                w_v[pl.ds(t * 16, 16)] = ex
                sdst_v[pl.ds(t * 16, 16)] = dv
                plsc.addupdate_scatter(den_v, [dv], ex)
            cp.wait()

            for t in range(CHUNK // 16):
                w16 = w_v[pl.ds(t * 16, 16)]
                for l in range(16):
                    w = w16[l]
                    k = t * 16 + l
                    for d in range(H // 32):
                        pk = rowsp_v[k, pl.ds(d * 16, 16)]
                        pi = plsc.bitcast(pk, jnp.int32)
                        lo = plsc.bitcast(jnp.left_shift(pi, 16),
                                          jnp.float32)
                        hi = plsc.bitcast(pi & mask_hi, jnp.float32)
                        rows_v[k, pl.ds(d * 32, 16)] = lo * w
                        rows_v[k, pl.ds(d * 32 + 16, 16)] = hi * w

            pltpu.sync_copy(rows_v, acc_sh.at[sdst_v], add=True)
            return carry2

        lax.fori_loop(0, SUPER, _chunk_body, 0)
        return carry

    lax.fori_loop(0, NSUPER, _super_body, 0)
    plsc.subcore_barrier()

    pltpu.sync_copy(den_v, den_out.at[wid])
    pltpu.sync_copy(acc_sh.at[pl.ds(sid * ROWS_MAIN, ROWS_MAIN)],
                    acc_out.at[cid, pl.ds(sid * ROWS_MAIN, ROWS_MAIN)])

    @pl.when(sid == 0)
    def _export_tail():
        pltpu.sync_copy(acc_sh.at[pl.ds(ROWS_TAIL_OFF, EXP_TAIL)],
                        acc_out.at[cid, pl.ds(ROWS_TAIL_OFF, EXP_TAIL)])


@functools.partial(
    pl.kernel,
    mesh=plsc.VectorSubcoreMesh(core_axis_name="c", subcore_axis_name="s"),
    compiler_params=pltpu.CompilerParams(needs_layout_passes=False,
                                         use_tc_tiling_on_sc=False),
    out_type=[
        jax.ShapeDtypeStruct((NC, N, H), jnp.float32),
        jax.ShapeDtypeStruct((NW, NP), jnp.float32),
    ],
    scratch_types=[
        pltpu.VMEM((NP,), jnp.float32),
        pltpu.VMEM((NP,), jnp.float32),
        pltpu.VMEM((16,), jnp.float32),
        pltpu.VMEM((NP,), jnp.float32),
        pltpu.VMEM((SUPER * CHUNK,), jnp.int32),
        pltpu.VMEM((SUPER * CHUNK,), jnp.int32),
        pltpu.VMEM((CHUNK,), jnp.int32),
        pltpu.VMEM((CHUNK,), jnp.int32),
        pltpu.VMEM((CHUNK,), jnp.float32),
        pltpu.VMEM((CHUNK, H // 2), jnp.float32),
        pltpu.VMEM((CHUNK, H), jnp.float32),
        pltpu.VMEM_SHARED((NP, H), jnp.float32),
        pltpu.SemaphoreType.DMA,
    ],
)
def _sc_gat(h_hbm, src_hbm, dst_hbm, a_hbm, b_hbm, m_hbm, zeros_hbm,
            acc_out, den_out, *scratch):
    _sc_body(h_hbm, src_hbm, dst_hbm, a_hbm, b_hbm, m_hbm, zeros_hbm,
             acc_out, den_out, *scratch)


def kernel(x, edge_index, batch, cat_features, W0, att_src0, att_dst0, bias0,
           gamma0, beta0, W1, att_src1, att_dst1, bias1, gamma1, beta1,
           W_cat, b_cat, W_lin, b_lin):
    src2 = edge_index[0].reshape(NW, EDGES_PER_TILE)
    dst2 = edge_index[1].reshape(NW, EDGES_PER_TILE)
    pad_src = jnp.zeros((NW, PADE), jnp.int32)
    pad_dst = jnp.broadcast_to(
        N + (jnp.arange(PADE, dtype=jnp.int32) % 8), (NW, PADE))
    src = jnp.concatenate([src2, pad_src], axis=1).reshape(-1)
    dst = jnp.concatenate([dst2, pad_dst], axis=1).reshape(-1)
    zpad = jnp.zeros((NP - N,), jnp.float32)
    zeros = jnp.zeros((NP, H), jnp.float32)

    def _pack_rows(hW):
        hb = hW.reshape(N, H // 32, 2, 16).astype(jnp.bfloat16)
        ht = jnp.transpose(hb, (0, 1, 3, 2))
        return lax.bitcast_convert_type(ht, jnp.float32).reshape(N, H // 2)

    hW, a, b, m, exs = _pre_layer(x, W0, att_src0, att_dst0)
    m16 = jnp.broadcast_to(m.reshape(1), (16,))
    ap = jnp.concatenate([a.reshape(N), zpad])
    bp = jnp.concatenate([b.reshape(N), zpad])
    acc, den = _sc_gat(_pack_rows(hW), src, dst, ap, bp, m16, zeros)

    hW, a, b, m, exs = _post_pre_layer(
        acc[0], acc[1], den.T[:N], hW, exs, bias0, gamma0, beta0,
        W1, att_src1, att_dst1)
    m16 = jnp.broadcast_to(m.reshape(1), (16,))
    ap = jnp.concatenate([a.reshape(N), zpad])
    bp = jnp.concatenate([b.reshape(N), zpad])
    acc, den = _sc_gat(_pack_rows(hW), src, dst, ap, bp, m16, zeros)

    return _post_final(acc[0], acc[1], den.T[:N], hW, exs, bias1, gamma1,
                       beta1, batch, cat_features, W_cat, b_cat, W_lin,
                       b_lin)


# speedup vs baseline: 1.5394x; 1.5394x over previous
import functools

import jax
import jax.numpy as jnp
from jax import lax
from jax.experimental import pallas as pl
from jax.experimental.pallas import tpu as pltpu
from jax.experimental.pallas import tpu_sc as plsc

N = 10000
E = 320000
H = 128
G = 64
CAT = 6

NC = 2
NS = 16
NW = NC * NS
EDGES_PER_TILE = E // NW
CHUNK = 80
NCHUNK = EDGES_PER_TILE // CHUNK
SUPER = 25
NSUPER = NCHUNK // SUPER
ROWS_MAIN = 624
ROWS_TAIL_OFF = NS * ROWS_MAIN
ROWS_TAIL = N - ROWS_TAIL_OFF


def _pre_body(x_ref, w_ref, as_ref, ad_ref, h_ref, a_ref, b_ref, m_ref, exs_ref):
    h = jnp.dot(x_ref[...], w_ref[...], preferred_element_type=jnp.float32)
    h_ref[...] = h
    a = jnp.dot(h, as_ref[...].reshape(H, 1), preferred_element_type=jnp.float32)
    b = jnp.dot(h, ad_ref[...].reshape(H, 1), preferred_element_type=jnp.float32)
    m = jnp.max(a)
    m_ref[...] = jnp.full((1, 1), 0.0, jnp.float32) + m
    cb = m + b
    c = jnp.where(cb > 0, cb, 0.2 * cb)
    z = a + b
    zl = jnp.where(z > 0, z, 0.2 * z)
    a_ref[...] = a
    b_ref[...] = b
    exs_ref[...] = jnp.exp(zl - c)


def _pre_layer(x, W, att_src, att_dst):
    return pl.pallas_call(
        _pre_body,
        out_shape=[
            jax.ShapeDtypeStruct((N, H), jnp.float32),
            jax.ShapeDtypeStruct((N, 1), jnp.float32),
            jax.ShapeDtypeStruct((N, 1), jnp.float32),
            jax.ShapeDtypeStruct((1, 1), jnp.float32),
            jax.ShapeDtypeStruct((N, 1), jnp.float32),
        ],
    )(x, W, att_src, att_dst)


def _post_compute(acc0_ref, acc1_ref, dent_ref, h_ref, exs_ref, bias_ref,
                  gamma_ref, beta_ref):
    exs = exs_ref[...]
    num = acc0_ref[...] + acc1_ref[...] + exs * h_ref[...]
    den = jnp.sum(dent_ref[...], axis=1, keepdims=True) + exs + 1e-16
    y = num / den + bias_ref[...]
    mu = jnp.mean(y, axis=1, keepdims=True)
    yc = y - mu
    var = jnp.mean(yc * yc, axis=1, keepdims=True)
    y = yc * lax.rsqrt(var + 1e-5) * gamma_ref[...] + beta_ref[...]
    return jnp.maximum(y, 0.0)


def _post_pre_body(acc0_ref, acc1_ref, dent_ref, h_ref, exs_ref, bias_ref,
                   gamma_ref, beta_ref, w_ref, as_ref, ad_ref,
                   h1_ref, a_ref, b_ref, m_ref, exs1_ref):
    h1 = _post_compute(acc0_ref, acc1_ref, dent_ref, h_ref, exs_ref,
                       bias_ref, gamma_ref, beta_ref)
    hW = jnp.dot(h1, w_ref[...], preferred_element_type=jnp.float32)
    h1_ref[...] = hW
    a = jnp.dot(hW, as_ref[...].reshape(H, 1),
                preferred_element_type=jnp.float32)
    b = jnp.dot(hW, ad_ref[...].reshape(H, 1),
                preferred_element_type=jnp.float32)
    m = jnp.max(a)
    m_ref[...] = jnp.full((1, 1), 0.0, jnp.float32) + m
    cb = m + b
    c = jnp.where(cb > 0, cb, 0.2 * cb)
    z = a + b
    zl = jnp.where(z > 0, z, 0.2 * z)
    a_ref[...] = a
    b_ref[...] = b
    exs1_ref[...] = jnp.exp(zl - c)


def _post_pre_layer(acc0, acc1, den_t, h, exs, bias, gamma, beta,
                    W, att_src, att_dst):
    return pl.pallas_call(
        _post_pre_body,
        out_shape=[
            jax.ShapeDtypeStruct((N, H), jnp.float32),
            jax.ShapeDtypeStruct((N, 1), jnp.float32),
            jax.ShapeDtypeStruct((N, 1), jnp.float32),
            jax.ShapeDtypeStruct((1, 1), jnp.float32),
            jax.ShapeDtypeStruct((N, 1), jnp.float32),
        ],
    )(acc0, acc1, den_t, h, exs, bias.reshape(1, H), gamma.reshape(1, H),
      beta.reshape(1, H), W, att_src, att_dst)


def _post_final_body(acc0_ref, acc1_ref, dent_ref, h_ref, exs_ref, bias_ref,
                     gamma_ref, beta_ref, batch_ref, cat_ref, wcat_ref,
                     bcat_ref, wlin_ref, blin_ref, out_ref):
    h2 = _post_compute(acc0_ref, acc1_ref, dent_ref, h_ref, exs_ref,
                       bias_ref, gamma_ref, beta_ref)
    bi = batch_ref[...]
    oh = (bi == lax.broadcasted_iota(jnp.int32, (N, G), 1)).astype(jnp.float32)
    s = lax.dot_general(oh, h2, (((0,), (0,)), ((), ())),
                        preferred_element_type=jnp.float32)
    ones = jnp.ones((N, 1), dtype=jnp.float32)
    cnt = lax.dot_general(oh, ones, (((0,), (0,)), ((), ())),
                          preferred_element_type=jnp.float32)
    pooled = s / jnp.maximum(cnt, 1.0)
    cat_emb = jnp.dot(cat_ref[...], wcat_ref[...],
                      preferred_element_type=jnp.float32) + bcat_ref[...]
    cat_emb = jnp.maximum(cat_emb, 0.0)
    z = pooled + cat_emb
    out_ref[...] = jnp.dot(z, wlin_ref[...],
                           preferred_element_type=jnp.float32) + blin_ref[...]


def _post_final(acc0, acc1, den_t, h, exs, bias, gamma, beta, batch,
                cat_features, W_cat, b_cat, W_lin, b_lin):
    return pl.pallas_call(
        _post_final_body,
        out_shape=jax.ShapeDtypeStruct((G, H), jnp.float32),
    )(acc0, acc1, den_t, h, exs, bias.reshape(1, H), gamma.reshape(1, H),
      beta.reshape(1, H), batch.reshape(N, 1), cat_features.reshape(G, CAT),
      W_cat, b_cat.reshape(1, H), W_lin, b_lin.reshape(1, H))


def _sc_body(h_hbm, src_hbm, dst_hbm, a_hbm, b_hbm, m_hbm, zeros_hbm,
             acc_out, den_out,
             a_v, b_v, m_v, den_v, srcs_v, dsts_v, ssrc_v, sdst_v, w_v,
             rowsp_v, rows_v, acc_sh, sem):
    cid = lax.axis_index("c")
    sid = lax.axis_index("s")
    wid = cid * NS + sid

    pltpu.sync_copy(a_hbm, a_v)
    pltpu.sync_copy(b_hbm, b_v)
    pltpu.sync_copy(m_hbm, m_v)

    def _zero_body(i, carry):
        off = pl.multiple_of(i * 16, 16)
        den_v[pl.ds(off, 16)] = jnp.zeros((16,), jnp.float32)
        return carry
    lax.fori_loop(0, N // 16, _zero_body, 0)
    pltpu.sync_copy(zeros_hbm.at[pl.ds(sid * ROWS_MAIN, ROWS_MAIN)],
                    acc_sh.at[pl.ds(sid * ROWS_MAIN, ROWS_MAIN)])

    @pl.when(sid == 0)
    def _zero_tail():
        pltpu.sync_copy(zeros_hbm.at[pl.ds(ROWS_TAIL_OFF, ROWS_TAIL)],
                        acc_sh.at[pl.ds(ROWS_TAIL_OFF, ROWS_TAIL)])

    plsc.subcore_barrier()

    base = wid * EDGES_PER_TILE
    m16 = m_v[...]

    def _super_body(i, carry):
        off = pl.multiple_of(base + i * SUPER * CHUNK, 8)
        pltpu.sync_copy(src_hbm.at[pl.ds(off, SUPER * CHUNK)], srcs_v)
        pltpu.sync_copy(dst_hbm.at[pl.ds(off, SUPER * CHUNK)], dsts_v)

        def _chunk_body(ci, carry2):
            for t in range(CHUNK // 16):
                so = pl.multiple_of(ci * CHUNK + t * 16, 16)
                ssrc_v[pl.ds(t * 16, 16)] = srcs_v[pl.ds(so, 16)]
            cp = pltpu.async_copy(h_hbm.at[ssrc_v], rowsp_v, sem)

            for t in range(CHUNK // 16):
                so = pl.multiple_of(ci * CHUNK + t * 16, 16)
                sv = ssrc_v[pl.ds(t * 16, 16)]
                dv = dsts_v[pl.ds(so, 16)]
                ag = plsc.load_gather(a_v, [sv])
                bg = plsc.load_gather(b_v, [dv])
                cb = m16 + bg
                cg = jnp.where(cb > 0, cb, 0.2 * cb)
                z = ag + bg
                e = jnp.where(z > 0, z, 0.2 * z)
                ex = jnp.exp(e - cg)
                w_v[pl.ds(t * 16, 16)] = ex
                sdst_v[pl.ds(t * 16, 16)] = dv
                plsc.addupdate_scatter(den_v, [dv], ex)
            cp.wait()

            mask_hi = jnp.full((16,), -65536, jnp.int32)
            for t in range(CHUNK // 16):
                w16 = w_v[pl.ds(t * 16, 16)]
                for j in range(16):
                    w = w16[j]
                    k = t * 16 + j
                    for d in range(H // 32):
                        p = rowsp_v[k, pl.ds(d * 16, 16)]
                        pi = plsc.bitcast(p, jnp.int32)
                        lo = plsc.bitcast(jnp.left_shift(pi, 16),
                                          jnp.float32)
                        hi = plsc.bitcast(pi & mask_hi, jnp.float32)
                        rows_v[k, pl.ds(d * 32, 16)] = lo * w
                        rows_v[k, pl.ds(d * 32 + 16, 16)] = hi * w

            pltpu.sync_copy(rows_v, acc_sh.at[sdst_v], add=True)
            return carry2

        lax.fori_loop(0, SUPER, _chunk_body, 0)
        return carry

    lax.fori_loop(0, NSUPER, _super_body, 0)
    plsc.subcore_barrier()

    pltpu.sync_copy(den_v, den_out.at[wid])
    pltpu.sync_copy(acc_sh.at[pl.ds(sid * ROWS_MAIN, ROWS_MAIN)],
                    acc_out.at[cid, pl.ds(sid * ROWS_MAIN, ROWS_MAIN)])

    @pl.when(sid == 0)
    def _export_tail():
        pltpu.sync_copy(acc_sh.at[pl.ds(ROWS_TAIL_OFF, ROWS_TAIL)],
                        acc_out.at[cid, pl.ds(ROWS_TAIL_OFF, ROWS_TAIL)])


@functools.partial(
    pl.kernel,
    mesh=plsc.VectorSubcoreMesh(core_axis_name="c", subcore_axis_name="s"),
    compiler_params=pltpu.CompilerParams(needs_layout_passes=False,
                                         use_tc_tiling_on_sc=False),
    out_type=[
        jax.ShapeDtypeStruct((NC, N, H), jnp.float32),
        jax.ShapeDtypeStruct((NW, N), jnp.float32),
    ],
    scratch_types=[
        pltpu.VMEM((N,), jnp.float32),
        pltpu.VMEM((N,), jnp.float32),
        pltpu.VMEM((16,), jnp.float32),
        pltpu.VMEM((N,), jnp.float32),
        pltpu.VMEM((SUPER * CHUNK,), jnp.int32),
        pltpu.VMEM((SUPER * CHUNK,), jnp.int32),
        pltpu.VMEM((CHUNK,), jnp.int32),
        pltpu.VMEM((CHUNK,), jnp.int32),
        pltpu.VMEM((CHUNK,), jnp.float32),
        pltpu.VMEM((CHUNK, H // 2), jnp.float32),
        pltpu.VMEM((CHUNK, H), jnp.float32),
        pltpu.VMEM_SHARED((N, H), jnp.float32),
        pltpu.SemaphoreType.DMA,
    ],
)
def _sc_gat(h_hbm, src_hbm, dst_hbm, a_hbm, b_hbm, m_hbm, zeros_hbm,
            acc_out, den_out, *scratch):
    _sc_body(h_hbm, src_hbm, dst_hbm, a_hbm, b_hbm, m_hbm, zeros_hbm,
             acc_out, den_out, *scratch)


def kernel(x, edge_index, batch, cat_features, W0, att_src0, att_dst0, bias0,
           gamma0, beta0, W1, att_src1, att_dst1, bias1, gamma1, beta1,
           W_cat, b_cat, W_lin, b_lin):
    src = edge_index[0]
    dst = edge_index[1]
    zeros = jnp.zeros((N, H), jnp.float32)

    def _pack_rows(hW):
        hb = hW.reshape(N, H // 32, 2, 16).astype(jnp.bfloat16)
        ht = jnp.transpose(hb, (0, 1, 3, 2))
        return lax.bitcast_convert_type(ht, jnp.float32).reshape(N, H // 2)

    hW, a, b, m, exs = _pre_layer(x, W0, att_src0, att_dst0)
    m16 = jnp.broadcast_to(m.reshape(1), (16,))
    acc, den = _sc_gat(_pack_rows(hW), src, dst, a.reshape(N), b.reshape(N),
                       m16, zeros)

    hW, a, b, m, exs = _post_pre_layer(
        acc[0], acc[1], den.T, hW, exs, bias0, gamma0, beta0,
        W1, att_src1, att_dst1)
    m16 = jnp.broadcast_to(m.reshape(1), (16,))
    acc, den = _sc_gat(_pack_rows(hW), src, dst, a.reshape(N), b.reshape(N),
                       m16, zeros)

    return _post_final(acc[0], acc[1], den.T, hW, exs, bias1, gamma1, beta1,
                       batch, cat_features, W_cat, b_cat, W_lin, b_lin)
